# Initial kernel scaffold; baseline (speedup 1.0000x reference)
import jax
import jax.numpy as jnp
from jax.experimental import pallas as pl


def kernel(x, wte):
    raise NotImplementedError("write your pallas kernel here")


# speedup vs baseline: 3.0387x; 3.0387x over previous
import functools

import jax
import jax.numpy as jnp
import numpy as np
from jax import lax
from jax.experimental import pallas as pl
from jax.experimental.pallas import tpu as pltpu
from jax.experimental.pallas import tpu_sc as plsc

NC = 2
NS = 16
NW = NC * NS
L = 16


def _make_lookup(B: int, V: int, D: int, chunk: int):
  per_w = B // NW
  assert per_w % chunk == 0 and chunk % L == 0
  n_steps = per_w // chunk
  n_groups = chunk // L
  mesh = plsc.VectorSubcoreMesh(
      core_axis_name="c", subcore_axis_name="s", num_cores=NC,
      num_subcores=NS)

  @functools.partial(
      pl.kernel,
      out_type=jax.ShapeDtypeStruct((B * D,), jnp.float32),
      mesh=mesh,
      scratch_types=[
          pltpu.VMEM((chunk,), jnp.int32),
          pltpu.VMEM((chunk * D,), jnp.float32),
          pltpu.VMEM((V * D,), jnp.float32),
          pltpu.SemaphoreType.DMA,
      ],
      compiler_params=pltpu.CompilerParams(
          use_tc_tiling_on_sc=False, needs_layout_passes=False),
  )
  def lookup(x_hbm, wte_hbm, out_hbm, idx_v, out_c, tab_v, sem):
    wid = lax.axis_index("s") * NC + lax.axis_index("c")
    base = wid * per_w
    pltpu.sync_copy(wte_hbm, tab_v)
    iota = lax.iota(jnp.int32, L)
    iota_d = iota * D

    def group(g, carry):
      xg = idx_v[pl.ds(g * L, L)]
      src = xg * D
      dst = g * (L * D) + iota_d

      for d in range(D):
        vals = plsc.load_gather(tab_v, [src + d])
        plsc.store_scatter(out_c, [dst + d], vals)
      return carry

    def step(i, carry):
      off = base + i * chunk
      pltpu.sync_copy(x_hbm.at[pl.ds(off, chunk)], idx_v)
      lax.fori_loop(0, n_groups, group, 0)
      pltpu.sync_copy(out_c, out_hbm.at[pl.ds(off * D, chunk * D)])
      return carry

    lax.fori_loop(0, n_steps, step, 0)

  return lookup


def kernel(x, wte):
  B = x.shape[0] * x.shape[1]
  V, D = wte.shape
  out = _make_lookup(B, V, D, chunk=3200)(x.reshape(B), wte.reshape(V * D))
  return out.reshape(x.shape[0], x.shape[1], D)


# speedup vs baseline: 3.6845x; 1.2125x over previous
import functools

import jax
import jax.numpy as jnp
from jax import lax
from jax.experimental import pallas as pl
from jax.experimental.pallas import tpu as pltpu
from jax.experimental.pallas import tpu_sc as plsc

NC = 2
NS = 16
NW = NC * NS
L = 16


def _make_lookup(X0: int, X1: int, V: int, D: int, rows_per_step: int):
  B = X0 * X1
  per_w_rows = X0 // NW
  assert X0 % (NW * rows_per_step) == 0
  n_steps = per_w_rows // rows_per_step
  chunk = rows_per_step * X1
  assert chunk % L == 0
  n_groups = chunk // L
  mesh = plsc.VectorSubcoreMesh(
      core_axis_name="c", subcore_axis_name="s", num_cores=NC,
      num_subcores=NS)

  @functools.partial(
      pl.kernel,
      out_type=jax.ShapeDtypeStruct((X0, X1, D), jnp.float32),
      mesh=mesh,
      scratch_types=[
          pltpu.VMEM((chunk,), jnp.int32),
          pltpu.VMEM((rows_per_step, X1, D), jnp.float32),
          pltpu.VMEM((V * D,), jnp.float32),
          pltpu.SemaphoreType.DMA,
      ],
      compiler_params=pltpu.CompilerParams(
          use_tc_tiling_on_sc=False, needs_layout_passes=False),
  )
  def lookup(x_hbm, wte_hbm, out_hbm, idx_v, out_c, tab_v, sem):
    wid = lax.axis_index("s") * NC + lax.axis_index("c")
    base_row = wid * per_w_rows
    pltpu.sync_copy(wte_hbm, tab_v)
    iota = lax.iota(jnp.int32, L)
    zero = iota * 0

    def group(g, carry):
      xg = idx_v[pl.ds(g * L, L)]
      src = xg * D
      n16 = g * L + iota
      i16 = n16 // X1
      j16 = n16 - i16 * X1

      for d in range(D):
        vals = plsc.load_gather(tab_v, [src + d])
        plsc.store_scatter(out_c, [i16, j16, zero + d], vals)
      return carry

    def step(i, carry):
      row = base_row + i * rows_per_step
      pltpu.sync_copy(x_hbm.at[pl.ds(row * X1, chunk)], idx_v)
      lax.fori_loop(0, n_groups, group, 0)
      pltpu.sync_copy(out_c, out_hbm.at[pl.ds(row, rows_per_step)])
      return carry

    lax.fori_loop(0, n_steps, step, 0)

  return lookup


def kernel(x, wte):
  X0, X1 = x.shape
  V, D = wte.shape
  return _make_lookup(X0, X1, V, D, rows_per_step=8)(
      x.reshape(X0 * X1), wte.reshape(V * D))


# speedup vs baseline: 8.9344x; 2.4249x over previous
import functools

import jax
import jax.numpy as jnp
from jax import lax
from jax.experimental import pallas as pl
from jax.experimental.pallas import tpu as pltpu
from jax.experimental.pallas import tpu_sc as plsc

NC = 2
NS = 16
NW = NC * NS
L = 16


def _make_lookup(X0: int, X1: int, V: int, D: int):
  IT = X0 // 128
  JT = X1 // 8
  assert X0 % 128 == 0 and X1 % 8 == 0 and IT % NW == 0
  it_per_w = IT // NW
  mesh = plsc.VectorSubcoreMesh(
      core_axis_name="c", subcore_axis_name="s", num_cores=NC,
      num_subcores=NS)

  @functools.partial(
      pl.kernel,
      out_type=jax.ShapeDtypeStruct((D, JT, IT, 8, 128), jnp.float32),
      mesh=mesh,
      scratch_types=[
          pltpu.VMEM((128, X1), jnp.int32),
          pltpu.VMEM((D, 8, 128), jnp.float32),
          pltpu.VMEM((V * D,), jnp.float32),
          pltpu.SemaphoreType.DMA,
      ],
      compiler_params=pltpu.CompilerParams(
          use_tc_tiling_on_sc=False, needs_layout_passes=False),
  )
  def lookup(x_hbm, wte_hbm, out_hbm, xblk, out_t, tab_v, sem):
    wid = lax.axis_index("s") * NC + lax.axis_index("c")
    pltpu.sync_copy(wte_hbm, tab_v)
    iota = lax.iota(jnp.int32, L)
    zero = iota * 0

    def do_jt(jt, it):
      def group(g, carry):
        js = g // 8
        ig = g - js * 8
        jv = zero + (jt * 8 + js)
        i16 = ig * L + iota
        xg = plsc.load_gather(xblk, [i16, jv])
        src = xg * D
        for d in range(D):
          vals = plsc.load_gather(tab_v, [src + d])
          out_t[d, js, pl.ds(ig * L, L)] = vals
        return carry

      lax.fori_loop(0, 64, group, 0)
      pltpu.sync_copy(out_t, out_hbm.at[:, jt, it])
      return it

    def do_it(a, carry):
      it = wid * it_per_w + a
      pltpu.sync_copy(x_hbm.at[pl.ds(it * 128, 128)], xblk)
      lax.fori_loop(0, JT, do_jt, it)
      return carry

    lax.fori_loop(0, it_per_w, do_it, 0)

  return lookup


def kernel(x, wte):
  X0, X1 = x.shape
  V, D = wte.shape
  out5 = _make_lookup(X0, X1, V, D)(x, wte.reshape(V * D))
  return out5.transpose(2, 4, 1, 3, 0).reshape(X0, X1, D)


# speedup vs baseline: 22.4729x; 2.5153x over previous
import functools

import jax
import jax.numpy as jnp
from jax import lax
from jax.experimental import pallas as pl
from jax.experimental.pallas import tpu as pltpu
from jax.experimental.pallas import tpu_sc as plsc

NC = 2
NS = 16
NW = NC * NS
L = 16


def _make_lookup(X0: int, X1: int, V: int, D: int):
  IT = X0 // 128
  JT = X1 // 8
  assert X0 % 128 == 0 and X1 % 8 == 0 and IT % NW == 0
  it_per_w = IT // NW
  mesh = plsc.VectorSubcoreMesh(
      core_axis_name="c", subcore_axis_name="s", num_cores=NC,
      num_subcores=NS)

  @functools.partial(
      pl.kernel,
      out_type=jax.ShapeDtypeStruct((D, JT, IT, 8, 128), jnp.float32),
      mesh=mesh,
      scratch_types=[
          pltpu.VMEM((128, X1), jnp.int32),
          pltpu.VMEM((D, 8, 128), jnp.float32),
          pltpu.VMEM((V * D,), jnp.float32),
          pltpu.SemaphoreType.DMA,
      ],
      compiler_params=pltpu.CompilerParams(
          use_tc_tiling_on_sc=False, needs_layout_passes=False),
  )
  def lookup(x_hbm, wte_hbm, out_hbm, xblk, out_t, tab_v, sem):
    wid = lax.axis_index("s") * NC + lax.axis_index("c")
    pltpu.sync_copy(wte_hbm, tab_v)
    iota = lax.iota(jnp.int32, L)
    zero = iota * 0

    def do_jt(jt, it):
      @plsc.parallel_loop(0, 64, unroll=2)
      def group(g):
        js = g // 8
        ig = g - js * 8
        jv = zero + (jt * 8 + js)
        i16 = ig * L + iota
        xg = plsc.load_gather(xblk, [i16, jv])
        src = xg * D
        for d in range(D):
          vals = plsc.load_gather(tab_v, [src + d])
          out_t[d, js, pl.ds(ig * L, L)] = vals
      pltpu.sync_copy(out_t, out_hbm.at[:, jt, it])
      return it

    def do_it(a, carry):
      it = wid * it_per_w + a
      pltpu.sync_copy(x_hbm.at[pl.ds(it * 128, 128)], xblk)
      lax.fori_loop(0, JT, do_jt, it)
      return carry

    lax.fori_loop(0, it_per_w, do_it, 0)

  return lookup


def kernel(x, wte):
  X0, X1 = x.shape
  V, D = wte.shape
  out5 = _make_lookup(X0, X1, V, D)(x, wte.reshape(V * D))
  return out5.transpose(2, 4, 1, 3, 0).reshape(X0, X1, D)
